# baseline (speedup 1.0000x reference)
import functools
import jax
import jax.numpy as jnp
import numpy as np
from jax.experimental import pallas as pl
from jax.experimental.pallas import tpu as pltpu

K = 16
DIMS = [32, 64, 128, 256]
N_LEVELS = 3


def _head_body(x_ref, w0, b0, w1, b1, w2, b2, w3, b3, o_ref):
    h = jnp.maximum(jnp.dot(x_ref[...], w0[...], preferred_element_type=jnp.float32) + b0[...], 0.0)
    h = jnp.maximum(jnp.dot(h, w1[...], preferred_element_type=jnp.float32) + b1[...], 0.0)
    h = jnp.maximum(jnp.dot(h, w2[...], preferred_element_type=jnp.float32) + b2[...], 0.0)
    o_ref[...] = jnp.dot(h, w3[...], preferred_element_type=jnp.float32) + b3[...]


def _head(params, x):
    n = x.shape[0]
    tile = 2048
    ws = []
    for p in params:
        ws.append(p["w"])
        ws.append(p["b"].reshape(1, -1))
    grid = (n // tile,)
    return pl.pallas_call(
        _head_body,
        grid=grid,
        in_specs=[pl.BlockSpec((tile, x.shape[1]), lambda i: (i, 0))]
        + [pl.BlockSpec(w.shape, lambda i: (0,) * w.ndim) for w in ws],
        out_specs=pl.BlockSpec((tile, 13), lambda i: (i, 0)),
        out_shape=jax.ShapeDtypeStruct((n, 13), jnp.float32),
    )(x, *ws)


def _knn(query, base, k, exclude_self=False, chunk=2048):
    b2 = jnp.sum(base * base, axis=-1)
    out = []
    nq = query.shape[0]
    for s in range(0, nq, chunk):
        q = query[s:s + chunk]
        d = jnp.sum(q * q, axis=-1)[:, None] + b2[None, :] - 2.0 * (q @ base.T)
        if exclude_self:
            r = jnp.arange(q.shape[0])
            d = d.at[r, s + r].set(jnp.inf)
        _, idx = jax.lax.top_k(-d, k)
        out.append(idx)
    return jnp.concatenate(out, axis=0)


def _knn_graph(pos, k):
    idx = _knn(pos, pos, k, exclude_self=True)
    self_idx = jnp.arange(pos.shape[0], dtype=idx.dtype)[:, None]
    return jnp.concatenate([idx, self_idx], axis=1)


def _fps(pos, ratio):
    n = int(np.ceil(ratio * pos.shape[0]))
    d0 = jnp.sum((pos - pos[0]) ** 2, axis=-1)
    sel0 = jnp.zeros(n, dtype=jnp.int32)

    def body(i, carry):
        sel, d = carry
        j = jnp.argmax(d)
        sel = sel.at[i].set(j.astype(sel.dtype))
        d = jnp.minimum(d, jnp.sum((pos - pos[j]) ** 2, axis=-1))
        return sel, d

    sel, _ = jax.lax.fori_loop(1, n, body, (sel0, d0))
    return sel


def _build_indices(pos):
    pos = jax.lax.stop_gradient(pos)
    pos_l = [pos]
    nbr = [_knn_graph(pos, K)]
    sel, td, up = [], [], []
    for i in range(N_LEVELS):
        s = _fps(pos_l[i], 0.25)
        p_sub = pos_l[i][s]
        sel.append(s)
        td.append(_knn(p_sub, pos_l[i], K))
        nbr.append(_knn_graph(p_sub, K))
        pos_l.append(p_sub)
    for i in range(N_LEVELS):
        up.append(_knn(pos_l[i], pos_l[i + 1], 3))
    return {"sel": sel, "td_knn": td, "nbr": nbr, "up_knn": up}


def _mlp_bn(ps, x):
    for p in ps:
        x = x @ p["w"] + p["b"]
        m = jnp.mean(x, axis=0)
        v = jnp.var(x, axis=0)
        x = (x - m) / jnp.sqrt(v + 1e-5) * p["g"] + p["beta"]
        x = jax.nn.relu(x)
    return x


def _mlp_plain(ps, x):
    for p in ps:
        x = jax.nn.relu(x @ p["w"] + p["b"])
    return x


def _tblock(p, x, pos, nbr):
    x = jax.nn.relu(x @ p["lin_in"]["w"] + p["lin_in"]["b"])
    q = x @ p["w_dst"]
    s = x @ p["w_src"]
    v = x @ p["w_lin"]
    pd = pos[:, None, :] - pos[nbr]
    delta = _mlp_plain(p["pos_nn"], pd)
    a = q[:, None, :] - s[nbr] + delta
    a = _mlp_plain(p["attn_nn"], a)
    a = jax.nn.softmax(a, axis=1)
    x = jnp.sum(a * (v[nbr] + delta), axis=1)
    x = jax.nn.relu(x @ p["lin_out"]["w"] + p["lin_out"]["b"])
    return x


def kernel(x, pos, params):
    idx = _build_indices(pos)
    pos_l = [pos]
    for i in range(N_LEVELS):
        pos_l.append(pos_l[i][idx["sel"][i]])
    x = _mlp_bn(params["mlp_input"], x)
    x = _tblock(params["t_in"], x, pos_l[0], idx["nbr"][0])
    outs = [x]
    for i in range(N_LEVELS):
        x = _mlp_bn(params["td"][i], x)
        x = jnp.max(x[idx["td_knn"][i]], axis=1)
        x = _tblock(params["tdown"][i], x, pos_l[i + 1], idx["nbr"][i + 1])
        outs.append(x)
    x = _mlp_plain(params["mlp_summit"], x)
    x = _tblock(params["t_summit"], x, pos_l[N_LEVELS], idx["nbr"][N_LEVELS])
    for i in range(N_LEVELS - 1, -1, -1):
        x_sub = _mlp_bn(params["tu"][i]["mlp_sub"], x)
        nbr3 = idx["up_knn"][i]
        diff = pos_l[i][:, None, :] - pos_l[i + 1][nbr3]
        d2 = jnp.clip(jnp.sum(diff * diff, axis=-1), 1e-16, None)
        w = jax.lax.stop_gradient(1.0 / d2)[:, :, None]
        x_int = jnp.sum(x_sub[nbr3] * w, axis=1) / jnp.sum(w, axis=1)
        x = _mlp_bn(params["tu"][i]["mlp"], outs[i]) + x_int
        x = _tblock(params["tup"][i], x, pos_l[i], idx["nbr"][i])
    return _head(params["head"], x)


# speedup vs baseline: 1.7443x; 1.7443x over previous
import functools
import jax
import jax.numpy as jnp
import numpy as np
from jax.experimental import pallas as pl
from jax.experimental.pallas import tpu as pltpu

K = 16
DIMS = [32, 64, 128, 256]
N_LEVELS = 3


def _head_body(x_ref, w0, b0, w1, b1, w2, b2, w3, b3, o_ref):
    h = jnp.maximum(jnp.dot(x_ref[...], w0[...], preferred_element_type=jnp.float32) + b0[...], 0.0)
    h = jnp.maximum(jnp.dot(h, w1[...], preferred_element_type=jnp.float32) + b1[...], 0.0)
    h = jnp.maximum(jnp.dot(h, w2[...], preferred_element_type=jnp.float32) + b2[...], 0.0)
    o_ref[...] = jnp.dot(h, w3[...], preferred_element_type=jnp.float32) + b3[...]


def _head(params, x):
    n = x.shape[0]
    tile = 2048
    ws = []
    for p in params:
        ws.append(p["w"])
        ws.append(p["b"].reshape(1, -1))
    grid = (n // tile,)
    return pl.pallas_call(
        _head_body,
        grid=grid,
        in_specs=[pl.BlockSpec((tile, x.shape[1]), lambda i: (i, 0))]
        + [pl.BlockSpec(w.shape, lambda i: (0,) * w.ndim) for w in ws],
        out_specs=pl.BlockSpec((tile, 13), lambda i: (i, 0)),
        out_shape=jax.ShapeDtypeStruct((n, 13), jnp.float32),
    )(x, *ws)


def _knn(query, base, k, exclude_self=False, chunk=2048):
    b2 = jnp.sum(base * base, axis=-1)
    out = []
    nq = query.shape[0]
    for s in range(0, nq, chunk):
        q = query[s:s + chunk]
        d = jnp.sum(q * q, axis=-1)[:, None] + b2[None, :] - 2.0 * (q @ base.T)
        if exclude_self:
            r = jnp.arange(q.shape[0])
            d = d.at[r, s + r].set(jnp.inf)
        _, idx = jax.lax.top_k(-d, k)
        out.append(idx)
    return jnp.concatenate(out, axis=0)


def _knn_graph(pos, k):
    idx = _knn(pos, pos, k, exclude_self=True)
    self_idx = jnp.arange(pos.shape[0], dtype=idx.dtype)[:, None]
    return jnp.concatenate([idx, self_idx], axis=1)


def _fps_body(xs_ref, ys_ref, zs_ref, sel_ref, nsel):
    x = xs_ref[...]
    y = ys_ref[...]
    z = zs_ref[...]
    rows, lanes = x.shape
    n = rows * lanes
    gidx = jax.lax.broadcasted_iota(jnp.int32, (rows, lanes), 0) * lanes + \
        jax.lax.broadcasted_iota(jnp.int32, (rows, lanes), 1)
    x0 = xs_ref[0, 0]
    y0 = ys_ref[0, 0]
    z0 = zs_ref[0, 0]
    d0 = (x - x0) ** 2 + (y - y0) ** 2 + (z - z0) ** 2
    sel_ref[0:1, 0:1] = jnp.zeros((1, 1), jnp.int32)

    def body(i, d):
        m = jnp.max(d)
        j = jnp.min(jnp.where(d == m, gidx, n))
        sel_ref[pl.ds(i, 1), :] = jnp.full((1, 1), j, jnp.int32)
        onehot = (gidx == j).astype(jnp.float32)
        xj = jnp.sum(onehot * x)
        yj = jnp.sum(onehot * y)
        zj = jnp.sum(onehot * z)
        dj = (x - xj) ** 2 + (y - yj) ** 2 + (z - zj) ** 2
        return jnp.minimum(d, dj)

    jax.lax.fori_loop(1, nsel, body, d0)


def _fps_pallas(pos):
    n = pos.shape[0]
    nsel = int(np.ceil(0.25 * n))
    rows = n // 128
    xs = pos[:, 0].reshape(rows, 128)
    ys = pos[:, 1].reshape(rows, 128)
    zs = pos[:, 2].reshape(rows, 128)
    sel = pl.pallas_call(
        functools.partial(_fps_body, nsel=nsel),
        in_specs=[pl.BlockSpec(xs.shape, lambda: (0, 0))] * 3,
        out_specs=pl.BlockSpec((nsel, 1), lambda: (0, 0)),
        out_shape=jax.ShapeDtypeStruct((nsel, 1), jnp.int32),
    )(xs, ys, zs)
    return sel.reshape(nsel)


def _build_indices(pos):
    pos = jax.lax.stop_gradient(pos)
    pos_l = [pos]
    nbr = [_knn_graph(pos, K)]
    sel, td, up = [], [], []
    for i in range(N_LEVELS):
        s = _fps_pallas(pos_l[i])
        p_sub = pos_l[i][s]
        sel.append(s)
        td.append(_knn(p_sub, pos_l[i], K))
        nbr.append(_knn_graph(p_sub, K))
        pos_l.append(p_sub)
    for i in range(N_LEVELS):
        up.append(_knn(pos_l[i], pos_l[i + 1], 3))
    return {"sel": sel, "td_knn": td, "nbr": nbr, "up_knn": up}


def _mlp_bn(ps, x):
    for p in ps:
        x = x @ p["w"] + p["b"]
        m = jnp.mean(x, axis=0)
        v = jnp.var(x, axis=0)
        x = (x - m) / jnp.sqrt(v + 1e-5) * p["g"] + p["beta"]
        x = jax.nn.relu(x)
    return x


def _mlp_plain(ps, x):
    for p in ps:
        x = jax.nn.relu(x @ p["w"] + p["b"])
    return x


def _tblock(p, x, pos, nbr):
    x = jax.nn.relu(x @ p["lin_in"]["w"] + p["lin_in"]["b"])
    q = x @ p["w_dst"]
    s = x @ p["w_src"]
    v = x @ p["w_lin"]
    pd = pos[:, None, :] - pos[nbr]
    delta = _mlp_plain(p["pos_nn"], pd)
    a = q[:, None, :] - s[nbr] + delta
    a = _mlp_plain(p["attn_nn"], a)
    a = jax.nn.softmax(a, axis=1)
    x = jnp.sum(a * (v[nbr] + delta), axis=1)
    x = jax.nn.relu(x @ p["lin_out"]["w"] + p["lin_out"]["b"])
    return x


def kernel(x, pos, params):
    idx = _build_indices(pos)
    pos_l = [pos]
    for i in range(N_LEVELS):
        pos_l.append(pos_l[i][idx["sel"][i]])
    x = _mlp_bn(params["mlp_input"], x)
    x = _tblock(params["t_in"], x, pos_l[0], idx["nbr"][0])
    outs = [x]
    for i in range(N_LEVELS):
        x = _mlp_bn(params["td"][i], x)
        x = jnp.max(x[idx["td_knn"][i]], axis=1)
        x = _tblock(params["tdown"][i], x, pos_l[i + 1], idx["nbr"][i + 1])
        outs.append(x)
    x = _mlp_plain(params["mlp_summit"], x)
    x = _tblock(params["t_summit"], x, pos_l[N_LEVELS], idx["nbr"][N_LEVELS])
    for i in range(N_LEVELS - 1, -1, -1):
        x_sub = _mlp_bn(params["tu"][i]["mlp_sub"], x)
        nbr3 = idx["up_knn"][i]
        diff = pos_l[i][:, None, :] - pos_l[i + 1][nbr3]
        d2 = jnp.clip(jnp.sum(diff * diff, axis=-1), 1e-16, None)
        w = jax.lax.stop_gradient(1.0 / d2)[:, :, None]
        x_int = jnp.sum(x_sub[nbr3] * w, axis=1) / jnp.sum(w, axis=1)
        x = _mlp_bn(params["tu"][i]["mlp"], outs[i]) + x_int
        x = _tblock(params["tup"][i], x, pos_l[i], idx["nbr"][i])
    return _head(params["head"], x)


# speedup vs baseline: 5.0739x; 2.9088x over previous
import functools
import jax
import jax.numpy as jnp
import numpy as np
from jax.experimental import pallas as pl
from jax.experimental.pallas import tpu as pltpu

K = 16
DIMS = [32, 64, 128, 256]
N_LEVELS = 3


def _head_body(x_ref, w0, b0, w1, b1, w2, b2, w3, b3, o_ref):
    h = jnp.maximum(jnp.dot(x_ref[...], w0[...], preferred_element_type=jnp.float32) + b0[...], 0.0)
    h = jnp.maximum(jnp.dot(h, w1[...], preferred_element_type=jnp.float32) + b1[...], 0.0)
    h = jnp.maximum(jnp.dot(h, w2[...], preferred_element_type=jnp.float32) + b2[...], 0.0)
    o_ref[...] = jnp.dot(h, w3[...], preferred_element_type=jnp.float32) + b3[...]


def _head(params, x):
    n = x.shape[0]
    tile = 2048
    ws = []
    for p in params:
        ws.append(p["w"])
        ws.append(p["b"].reshape(1, -1))
    grid = (n // tile,)
    return pl.pallas_call(
        _head_body,
        grid=grid,
        in_specs=[pl.BlockSpec((tile, x.shape[1]), lambda i: (i, 0))]
        + [pl.BlockSpec(w.shape, lambda i: (0,) * w.ndim) for w in ws],
        out_specs=pl.BlockSpec((tile, 13), lambda i: (i, 0)),
        out_shape=jax.ShapeDtypeStruct((n, 13), jnp.float32),
    )(x, *ws)


def _knn_body(qp_ref, bT_ref, out_ref, d_scr, k, excl, tq):
    step = pl.program_id(0)
    nb = bT_ref.shape[1]
    nblk = nb // 128
    qp = qp_ref[...]
    bT = bT_ref[...]
    qb = jnp.dot(qp, bT, preferred_element_type=jnp.float32)
    q2 = jnp.sum(qp * qp, axis=1, keepdims=True)
    b2 = jnp.sum(bT * bT, axis=0, keepdims=True)
    d = (q2 + b2) - 2.0 * qb
    if excl:
        col = jax.lax.broadcasted_iota(jnp.int32, (tq, nb), 1)
        row = jax.lax.broadcasted_iota(jnp.int32, (tq, nb), 0) + step * tq
        d = jnp.where(col == row, 1e30, d)

    if nblk > k:
        d_scr[...] = d
        d3 = d.reshape(tq, nblk, 128)
        M = jnp.min(d3, axis=2)
        blk_iota = jax.lax.broadcasted_iota(jnp.int32, (tq, nblk), 1)
        bs = []
        for _ in range(k):
            bmin = jnp.min(M, axis=1, keepdims=True)
            bidx = jnp.min(jnp.where(M == bmin, blk_iota, nblk), axis=1, keepdims=True)
            bs.append(bidx)
            M = jnp.where(blk_iota == bidx, jnp.inf, M)
        B = jnp.concatenate(bs, axis=1)
        CH = 8
        c_q = jax.lax.broadcasted_iota(jnp.int32, (CH, k, CH * nblk), 2) // nblk
        c_b = jax.lax.broadcasted_iota(jnp.int32, (CH, k, CH * nblk), 2) % nblk
        q_i = jax.lax.broadcasted_iota(jnp.int32, (CH, k, CH * nblk), 0)
        w_i = jax.lax.broadcasted_iota(jnp.int32, (CH, k, 128), 2)
        Cs, Gs = [], []
        for qc in range(tq // CH):
            Bc3 = B[qc * CH:(qc + 1) * CH][:, :, None]
            oh = ((c_b == Bc3) & (c_q == q_i)).astype(jnp.float32)
            oh = oh.reshape(CH * k, CH * nblk)
            d2c = d_scr[qc * CH:(qc + 1) * CH, :].reshape(CH * nblk, 128)
            cc = jnp.dot(oh, d2c, preferred_element_type=jnp.float32,
                         precision=jax.lax.Precision.HIGHEST)
            Cs.append(cc.reshape(CH, k, 128))
            Gs.append(Bc3 * 128 + w_i)
        C = jnp.concatenate(Cs, axis=0).reshape(tq, k * 128)
        G = jnp.concatenate(Gs, axis=0).reshape(tq, k * 128)
    else:
        C = d
        G = jax.lax.broadcasted_iota(jnp.int32, (tq, nb), 1)

    outs = []
    for _ in range(k):
        m = jnp.min(C, axis=1, keepdims=True)
        ii = jnp.min(jnp.where(C == m, G, nb), axis=1, keepdims=True)
        outs.append(ii)
        C = jnp.where(G == ii, jnp.inf, C)
    out_ref[...] = jnp.concatenate(outs, axis=1)


def _knn_pallas(query, base, k, exclude_self=False):
    nq, nb = query.shape[0], base.shape[0]
    tq = min(nq, 128)
    qp = jnp.pad(query, ((0, 0), (0, 5)))
    bT = jnp.pad(base, ((0, 0), (0, 5))).T
    grid = (nq // tq,)
    return pl.pallas_call(
        functools.partial(_knn_body, k=k, excl=exclude_self, tq=tq),
        grid=grid,
        in_specs=[
            pl.BlockSpec((tq, 8), lambda i: (i, 0)),
            pl.BlockSpec((8, nb), lambda i: (0, 0)),
        ],
        out_specs=pl.BlockSpec((tq, k), lambda i: (i, 0)),
        out_shape=jax.ShapeDtypeStruct((nq, k), jnp.int32),
        scratch_shapes=[pltpu.VMEM((tq, nb), jnp.float32)],
    )(qp, bT)


def _knn(query, base, k, exclude_self=False, chunk=2048):
    return _knn_pallas(query, base, k, exclude_self=exclude_self)


def _knn_graph(pos, k):
    idx = _knn(pos, pos, k, exclude_self=True)
    self_idx = jnp.arange(pos.shape[0], dtype=idx.dtype)[:, None]
    return jnp.concatenate([idx, self_idx], axis=1)


def _fps_body(xs_ref, ys_ref, zs_ref, sel_ref, nsel):
    x = xs_ref[...]
    y = ys_ref[...]
    z = zs_ref[...]
    rows, lanes = x.shape
    n = rows * lanes
    gidx = jax.lax.broadcasted_iota(jnp.int32, (rows, lanes), 0) * lanes + \
        jax.lax.broadcasted_iota(jnp.int32, (rows, lanes), 1)
    x0 = xs_ref[0, 0]
    y0 = ys_ref[0, 0]
    z0 = zs_ref[0, 0]
    d0 = (x - x0) ** 2 + (y - y0) ** 2 + (z - z0) ** 2
    sel_ref[0:1, 0:1] = jnp.zeros((1, 1), jnp.int32)

    def body(i, d):
        m = jnp.max(d)
        j = jnp.min(jnp.where(d == m, gidx, n))
        sel_ref[pl.ds(i, 1), :] = jnp.full((1, 1), j, jnp.int32)
        onehot = (gidx == j).astype(jnp.float32)
        xj = jnp.sum(onehot * x)
        yj = jnp.sum(onehot * y)
        zj = jnp.sum(onehot * z)
        dj = (x - xj) ** 2 + (y - yj) ** 2 + (z - zj) ** 2
        return jnp.minimum(d, dj)

    jax.lax.fori_loop(1, nsel, body, d0)


def _fps_pallas(pos):
    n = pos.shape[0]
    nsel = int(np.ceil(0.25 * n))
    rows = n // 128
    xs = pos[:, 0].reshape(rows, 128)
    ys = pos[:, 1].reshape(rows, 128)
    zs = pos[:, 2].reshape(rows, 128)
    sel = pl.pallas_call(
        functools.partial(_fps_body, nsel=nsel),
        in_specs=[pl.BlockSpec(xs.shape, lambda: (0, 0))] * 3,
        out_specs=pl.BlockSpec((nsel, 1), lambda: (0, 0)),
        out_shape=jax.ShapeDtypeStruct((nsel, 1), jnp.int32),
    )(xs, ys, zs)
    return sel.reshape(nsel)


def _build_indices(pos):
    pos = jax.lax.stop_gradient(pos)
    pos_l = [pos]
    nbr = [_knn_graph(pos, K)]
    sel, td, up = [], [], []
    for i in range(N_LEVELS):
        s = _fps_pallas(pos_l[i])
        p_sub = pos_l[i][s]
        sel.append(s)
        td.append(_knn(p_sub, pos_l[i], K))
        nbr.append(_knn_graph(p_sub, K))
        pos_l.append(p_sub)
    for i in range(N_LEVELS):
        up.append(_knn(pos_l[i], pos_l[i + 1], 3))
    return {"sel": sel, "td_knn": td, "nbr": nbr, "up_knn": up}


def _mlp_bn(ps, x):
    for p in ps:
        x = x @ p["w"] + p["b"]
        m = jnp.mean(x, axis=0)
        v = jnp.var(x, axis=0)
        x = (x - m) / jnp.sqrt(v + 1e-5) * p["g"] + p["beta"]
        x = jax.nn.relu(x)
    return x


def _mlp_plain(ps, x):
    for p in ps:
        x = jax.nn.relu(x @ p["w"] + p["b"])
    return x


def _tblock(p, x, pos, nbr):
    x = jax.nn.relu(x @ p["lin_in"]["w"] + p["lin_in"]["b"])
    q = x @ p["w_dst"]
    s = x @ p["w_src"]
    v = x @ p["w_lin"]
    pd = pos[:, None, :] - pos[nbr]
    delta = _mlp_plain(p["pos_nn"], pd)
    a = q[:, None, :] - s[nbr] + delta
    a = _mlp_plain(p["attn_nn"], a)
    a = jax.nn.softmax(a, axis=1)
    x = jnp.sum(a * (v[nbr] + delta), axis=1)
    x = jax.nn.relu(x @ p["lin_out"]["w"] + p["lin_out"]["b"])
    return x


def kernel(x, pos, params):
    idx = _build_indices(pos)
    pos_l = [pos]
    for i in range(N_LEVELS):
        pos_l.append(pos_l[i][idx["sel"][i]])
    x = _mlp_bn(params["mlp_input"], x)
    x = _tblock(params["t_in"], x, pos_l[0], idx["nbr"][0])
    outs = [x]
    for i in range(N_LEVELS):
        x = _mlp_bn(params["td"][i], x)
        x = jnp.max(x[idx["td_knn"][i]], axis=1)
        x = _tblock(params["tdown"][i], x, pos_l[i + 1], idx["nbr"][i + 1])
        outs.append(x)
    x = _mlp_plain(params["mlp_summit"], x)
    x = _tblock(params["t_summit"], x, pos_l[N_LEVELS], idx["nbr"][N_LEVELS])
    for i in range(N_LEVELS - 1, -1, -1):
        x_sub = _mlp_bn(params["tu"][i]["mlp_sub"], x)
        nbr3 = idx["up_knn"][i]
        diff = pos_l[i][:, None, :] - pos_l[i + 1][nbr3]
        d2 = jnp.clip(jnp.sum(diff * diff, axis=-1), 1e-16, None)
        w = jax.lax.stop_gradient(1.0 / d2)[:, :, None]
        x_int = jnp.sum(x_sub[nbr3] * w, axis=1) / jnp.sum(w, axis=1)
        x = _mlp_bn(params["tu"][i]["mlp"], outs[i]) + x_int
        x = _tblock(params["tup"][i], x, pos_l[i], idx["nbr"][i])
    return _head(params["head"], x)


# speedup vs baseline: 8.6079x; 1.6965x over previous
import functools
import jax
import jax.numpy as jnp
import numpy as np
from jax.experimental import pallas as pl
from jax.experimental.pallas import tpu as pltpu

K = 16
DIMS = [32, 64, 128, 256]
N_LEVELS = 3


def _head_body(x_ref, w0, b0, w1, b1, w2, b2, w3, b3, o_ref):
    h = jnp.maximum(jnp.dot(x_ref[...], w0[...], preferred_element_type=jnp.float32) + b0[...], 0.0)
    h = jnp.maximum(jnp.dot(h, w1[...], preferred_element_type=jnp.float32) + b1[...], 0.0)
    h = jnp.maximum(jnp.dot(h, w2[...], preferred_element_type=jnp.float32) + b2[...], 0.0)
    o_ref[...] = jnp.dot(h, w3[...], preferred_element_type=jnp.float32) + b3[...]


def _head(params, x):
    n = x.shape[0]
    tile = 2048
    ws = []
    for p in params:
        ws.append(p["w"])
        ws.append(p["b"].reshape(1, -1))
    grid = (n // tile,)
    return pl.pallas_call(
        _head_body,
        grid=grid,
        in_specs=[pl.BlockSpec((tile, x.shape[1]), lambda i: (i, 0))]
        + [pl.BlockSpec(w.shape, lambda i: (0,) * w.ndim) for w in ws],
        out_specs=pl.BlockSpec((tile, 13), lambda i: (i, 0)),
        out_shape=jax.ShapeDtypeStruct((n, 13), jnp.float32),
    )(x, *ws)


def _knn_body(qp_ref, bT_ref, out_ref, d_scr, k, excl, tq):
    step = pl.program_id(0)
    nb = bT_ref.shape[1]
    nblk = nb // 128
    qp = qp_ref[...]
    bT = bT_ref[...]
    qb = jnp.dot(qp, bT, preferred_element_type=jnp.float32)
    q2 = jnp.sum(qp * qp, axis=1, keepdims=True)
    b2 = jnp.sum(bT * bT, axis=0, keepdims=True)
    d = (q2 + b2) - 2.0 * qb
    if excl:
        col = jax.lax.broadcasted_iota(jnp.int32, (tq, nb), 1)
        row = jax.lax.broadcasted_iota(jnp.int32, (tq, nb), 0) + step * tq
        d = jnp.where(col == row, 1e30, d)

    if nblk > k:
        d_scr[...] = d
        d3 = d.reshape(tq, nblk, 128)
        M = jnp.min(d3, axis=2)
        blk_iota = jax.lax.broadcasted_iota(jnp.int32, (tq, nblk), 1)
        bs = []
        for _ in range(k):
            bmin = jnp.min(M, axis=1, keepdims=True)
            bidx = jnp.min(jnp.where(M == bmin, blk_iota, nblk), axis=1, keepdims=True)
            bs.append(bidx)
            M = jnp.where(blk_iota == bidx, jnp.inf, M)
        B = jnp.concatenate(bs, axis=1)
        CH = 8
        c_q = jax.lax.broadcasted_iota(jnp.int32, (CH, k, CH * nblk), 2) // nblk
        c_b = jax.lax.broadcasted_iota(jnp.int32, (CH, k, CH * nblk), 2) % nblk
        q_i = jax.lax.broadcasted_iota(jnp.int32, (CH, k, CH * nblk), 0)
        w_i = jax.lax.broadcasted_iota(jnp.int32, (CH, k, 128), 2)
        Cs, Gs = [], []
        for qc in range(tq // CH):
            Bc3 = B[qc * CH:(qc + 1) * CH][:, :, None]
            oh = ((c_b == Bc3) & (c_q == q_i)).astype(jnp.float32)
            oh = oh.reshape(CH * k, CH * nblk)
            d2c = d_scr[qc * CH:(qc + 1) * CH, :].reshape(CH * nblk, 128)
            cc = jnp.dot(oh, d2c, preferred_element_type=jnp.float32,
                         precision=jax.lax.Precision.HIGHEST)
            Cs.append(cc.reshape(CH, k, 128))
            Gs.append(Bc3 * 128 + w_i)
        C = jnp.concatenate(Cs, axis=0).reshape(tq, k * 128)
        G = jnp.concatenate(Gs, axis=0).reshape(tq, k * 128)
    else:
        C = d
        G = jax.lax.broadcasted_iota(jnp.int32, (tq, nb), 1)

    outs = []
    for _ in range(k):
        m = jnp.min(C, axis=1, keepdims=True)
        ii = jnp.min(jnp.where(C == m, G, nb), axis=1, keepdims=True)
        outs.append(ii)
        C = jnp.where(G == ii, jnp.inf, C)
    out_ref[...] = jnp.concatenate(outs, axis=1)


def _knn_pallas(query, base, k, exclude_self=False):
    nq, nb = query.shape[0], base.shape[0]
    tq = min(nq, 128)
    qp = jnp.pad(query, ((0, 0), (0, 5)))
    bT = jnp.pad(base, ((0, 0), (0, 5))).T
    grid = (nq // tq,)
    return pl.pallas_call(
        functools.partial(_knn_body, k=k, excl=exclude_self, tq=tq),
        grid=grid,
        in_specs=[
            pl.BlockSpec((tq, 8), lambda i: (i, 0)),
            pl.BlockSpec((8, nb), lambda i: (0, 0)),
        ],
        out_specs=pl.BlockSpec((tq, k), lambda i: (i, 0)),
        out_shape=jax.ShapeDtypeStruct((nq, k), jnp.int32),
        scratch_shapes=[pltpu.VMEM((tq, nb), jnp.float32)],
    )(qp, bT)


def _knn(query, base, k, exclude_self=False, chunk=2048):
    return _knn_pallas(query, base, k, exclude_self=exclude_self)


def _knn_graph(pos, k):
    idx = _knn(pos, pos, k, exclude_self=True)
    self_idx = jnp.arange(pos.shape[0], dtype=idx.dtype)[:, None]
    return jnp.concatenate([idx, self_idx], axis=1)


def _fps_body(xs_ref, ys_ref, zs_ref, sel_ref, nsel):
    x = xs_ref[...]
    y = ys_ref[...]
    z = zs_ref[...]
    rows, lanes = x.shape
    n = rows * lanes
    gidx = jax.lax.broadcasted_iota(jnp.int32, (rows, lanes), 0) * lanes + \
        jax.lax.broadcasted_iota(jnp.int32, (rows, lanes), 1)
    x0 = xs_ref[0, 0]
    y0 = ys_ref[0, 0]
    z0 = zs_ref[0, 0]
    d0 = (x - x0) ** 2 + (y - y0) ** 2 + (z - z0) ** 2
    sel_ref[0:1, 0:1] = jnp.zeros((1, 1), jnp.int32)

    def body(i, d):
        m = jnp.max(d)
        j = jnp.min(jnp.where(d == m, gidx, n))
        sel_ref[pl.ds(i, 1), :] = jnp.full((1, 1), j, jnp.int32)
        onehot = (gidx == j).astype(jnp.float32)
        xj = jnp.sum(onehot * x)
        yj = jnp.sum(onehot * y)
        zj = jnp.sum(onehot * z)
        dj = (x - xj) ** 2 + (y - yj) ** 2 + (z - zj) ** 2
        return jnp.minimum(d, dj)

    jax.lax.fori_loop(1, nsel, body, d0)


def _fps_pallas(pos):
    n = pos.shape[0]
    nsel = int(np.ceil(0.25 * n))
    rows = n // 128
    xs = pos[:, 0].reshape(rows, 128)
    ys = pos[:, 1].reshape(rows, 128)
    zs = pos[:, 2].reshape(rows, 128)
    sel = pl.pallas_call(
        functools.partial(_fps_body, nsel=nsel),
        in_specs=[pl.BlockSpec(xs.shape, lambda: (0, 0))] * 3,
        out_specs=pl.BlockSpec((nsel, 1), lambda: (0, 0)),
        out_shape=jax.ShapeDtypeStruct((nsel, 1), jnp.int32),
    )(xs, ys, zs)
    return sel.reshape(nsel)


def _build_indices(pos):
    pos = jax.lax.stop_gradient(pos)
    pos_l = [pos]
    nbr = [_knn_graph(pos, K)]
    sel, td, up = [], [], []
    for i in range(N_LEVELS):
        s = _fps_pallas(pos_l[i])
        p_sub = pos_l[i][s]
        sel.append(s)
        td.append(_knn(p_sub, pos_l[i], K))
        nbr.append(_knn_graph(p_sub, K))
        pos_l.append(p_sub)
    for i in range(N_LEVELS):
        up.append(_knn(pos_l[i], pos_l[i + 1], 3))
    return {"sel": sel, "td_knn": td, "nbr": nbr, "up_knn": up}


def _sc_gather_seg(table, idx):
    from jax.experimental.pallas import tpu_sc as plsc
    V, D = table.shape
    B = idx.shape[0]
    NW = 32
    bpw = B // NW
    ch = 8
    for c in range(8, min(bpw, 128) + 1, 8):
        if bpw % c == 0:
            ch = c
    nch = bpw // ch
    mesh = plsc.VectorSubcoreMesh(core_axis_name="c", subcore_axis_name="s")

    @functools.partial(
        pl.kernel, mesh=mesh,
        out_type=jax.ShapeDtypeStruct((B, D), jnp.float32),
        scratch_types=[
            pltpu.VMEM((ch,), jnp.int32),
            pltpu.VMEM((ch, D), jnp.float32),
            pltpu.SemaphoreType.DMA,
        ],
    )
    def k(table_hbm, idx_hbm, out_hbm, idx_v, rows_v, sem):
        wid = jax.lax.axis_index("s") * 2 + jax.lax.axis_index("c")
        base = wid * bpw
        for c in range(nch):
            off = base + c * ch
            pltpu.sync_copy(idx_hbm.at[pl.ds(off, ch)], idx_v)
            pltpu.async_copy(table_hbm.at[idx_v], rows_v, sem).wait()
            pltpu.sync_copy(rows_v, out_hbm.at[pl.ds(off, ch)])

    return k(table, idx)


def _sc_gather(table, idx):
    w = table.shape[1]
    if w % 128:
        table = jnp.pad(table, ((0, 0), (0, 128 - w % 128)))
    B = idx.shape[0]
    seg = 69632
    if B <= seg:
        return _sc_gather_seg(table, idx)
    parts = []
    for s in range(0, B, seg):
        parts.append(_sc_gather_seg(table, idx[s:s + seg]))
    return jnp.concatenate(parts, axis=0)


def _dot(a, b):
    return jnp.dot(a, b, preferred_element_type=jnp.float32)


def _bn_body(x_ref, w, b, g, beta, o_ref):
    y = _dot(x_ref[...], w[...]) + b[...]
    mu = jnp.mean(y, axis=0, keepdims=True)
    var = jnp.mean((y - mu) ** 2, axis=0, keepdims=True)
    o_ref[...] = jnp.maximum((y - mu) / jnp.sqrt(var + 1e-5) * g[...] + beta[...], 0.0)


def _bn_pallas(p, x):
    n, ci = x.shape
    co = p["w"].shape[1]
    return pl.pallas_call(
        _bn_body,
        out_shape=jax.ShapeDtypeStruct((n, co), jnp.float32),
    )(x, p["w"], p["b"].reshape(1, -1), p["g"].reshape(1, -1), p["beta"].reshape(1, -1))


def _qsv_body(refs, mode, c, nslab):
    slabs = refs[:nslab]
    pos_ref, wi, bi, wd, ws, wl = refs[nslab:nslab + 6]
    extra = refs[nslab + 6:-2]
    q_ref, tab_ref = refs[-2:]
    cin = wi.shape[0]
    if mode == "pool":
        x = slabs[0][...][:, 0:cin]
        for j in range(1, 16):
            x = jnp.maximum(x, slabs[j][...][:, 0:cin])
    elif mode == "interp":
        x_ref = slabs[0]
        cs = cin
        pos3 = pos_ref[...][:, 0:3]
        num = None
        den = None
        for j in range(3):
            gj = slabs[1 + j][...]
            diff = pos3 - gj[:, cs:cs + 3]
            d2 = jnp.sum(diff * diff, axis=1, keepdims=True)
            d2 = jnp.maximum(d2, 1e-16)
            w8 = 1.0 / d2
            contrib = gj[:, 0:cs] * w8
            num = contrib if num is None else num + contrib
            den = w8 if den is None else den + w8
        x = x_ref[...] + num / den
    elif mode == "pre":
        wp, bp = extra
        x = jnp.maximum(_dot(slabs[0][...], wp[...]) + bp[...], 0.0)
    else:
        x = slabs[0][...]
    h = jnp.maximum(_dot(x, wi[...]) + bi[...], 0.0)
    q_ref[...] = _dot(h, wd[...])
    tab_ref[:, 0:c] = _dot(h, ws[...])
    tab_ref[:, c:2 * c] = _dot(h, wl[...])
    tab_ref[:, 2 * c:2 * c + 16] = pos_ref[...]


def _qsv_body2(*refs, mode, c, nslab):
    _qsv_body(refs, mode, c, nslab)


def _qsv_pallas(p, x, pospad, mode="plain", g=None, pre=None):
    n = pospad.shape[0]
    c = p["w_dst"].shape[1]
    t = min(512, n)
    nt = n // t
    if mode == "pool":
        slab_arrs = [x] * 16
        slab_specs = [pl.BlockSpec((t, x.shape[1]),
                                   functools.partial(lambda i, jj: (jj * nt + i, 0), jj=j))
                      for j in range(16)]
    elif mode == "interp":
        slab_arrs = [x] + [g] * 3
        slab_specs = [pl.BlockSpec((t, x.shape[1]), lambda i: (i, 0))] + \
            [pl.BlockSpec((t, g.shape[1]),
                          functools.partial(lambda i, jj: (jj * nt + i, 0), jj=j))
             for j in range(3)]
    else:
        slab_arrs = [x]
        slab_specs = [pl.BlockSpec((t, x.shape[1]), lambda i: (i, 0))]
    nslab = len(slab_arrs)
    ws = [p["lin_in"]["w"], p["lin_in"]["b"].reshape(1, -1),
          p["w_dst"], p["w_src"], p["w_lin"]]
    extra_in = []
    if mode == "pre":
        extra_in = [pre["w"], pre["b"].reshape(1, -1)]
    return pl.pallas_call(
        functools.partial(_qsv_body2, mode=mode, c=c, nslab=nslab),
        grid=(nt,),
        in_specs=slab_specs
        + [pl.BlockSpec((t, 16), lambda i: (i, 0))]
        + [pl.BlockSpec(w.shape, lambda i: (0,) * w.ndim) for w in ws]
        + [pl.BlockSpec(w.shape, lambda i: (0,) * w.ndim) for w in extra_in],
        out_specs=(pl.BlockSpec((t, c), lambda i: (i, 0)),
                   pl.BlockSpec((t, 2 * c + 16), lambda i: (i, 0))),
        out_shape=(jax.ShapeDtypeStruct((n, c), jnp.float32),
                   jax.ShapeDtypeStruct((n, 2 * c + 16), jnp.float32)),
    )(*slab_arrs, pospad, *ws, *extra_in)


def _edge_body(refs, c, nn, t):
    gs = refs[:nn]
    q_ref, pos_ref = refs[nn], refs[nn + 1]
    w1, b1, w2, b2, a1, ab1, a2, ab2, wo, bo = refs[nn + 2:nn + 12]
    o_ref = refs[nn + 12]
    pos3 = pos_ref[...][:, 0:3]
    q = q_ref[...]

    def delta_j(j):
        gj = gs[j]
        pd = pos3 - gj[:, 2 * c:2 * c + 3]
        h = jnp.maximum(_dot(pd, w1[...]) + b1[...], 0.0)
        return jnp.maximum(_dot(h, w2[...]) + b2[...], 0.0)

    avs = []
    for j in range(nn):
        a = q - gs[j][:, 0:c] + delta_j(j)
        h = jnp.maximum(_dot(a, a1[...]) + ab1[...], 0.0)
        avs.append(jnp.maximum(_dot(h, a2[...]) + ab2[...], 0.0))
    mx = avs[0]
    for j in range(1, nn):
        mx = jnp.maximum(mx, avs[j])
    ssum = None
    for j in range(nn):
        e = jnp.exp(avs[j] - mx)
        ssum = e if ssum is None else ssum + e
    acc = None
    for j in range(nn):
        term = (jnp.exp(avs[j] - mx) / ssum) * (gs[j][:, c:2 * c] + delta_j(j))
        acc = term if acc is None else acc + term
    o_ref[...] = jnp.maximum(_dot(acc, wo[...]) + bo[...], 0.0)


def _edge_pallas(p, g, q, pospad, nn):
    n, c = q.shape
    wd = g.shape[1]
    row_bytes = nn * wd * 8 + 5 * nn * c * 4
    t = 64
    for cand in (1024, 512, 256, 128, 64):
        if n % cand == 0 and cand * row_bytes <= 16_000_000:
            t = min(cand, n)
            break
    grid = (n // t,)
    nt = n // t
    g_specs = [pl.BlockSpec((t, wd), functools.partial(lambda i, jj: (jj * nt + i, 0), jj=j))
               for j in range(nn)]
    ws = [p["pos_nn"][0]["w"], p["pos_nn"][0]["b"].reshape(1, -1),
          p["pos_nn"][1]["w"], p["pos_nn"][1]["b"].reshape(1, -1),
          p["attn_nn"][0]["w"], p["attn_nn"][0]["b"].reshape(1, -1),
          p["attn_nn"][1]["w"], p["attn_nn"][1]["b"].reshape(1, -1),
          p["lin_out"]["w"], p["lin_out"]["b"].reshape(1, -1)]
    return pl.pallas_call(
        functools.partial(_edge_body2, c=c, nn=nn, t=t),
        grid=grid,
        in_specs=g_specs
        + [pl.BlockSpec((t, c), lambda i: (i, 0)),
           pl.BlockSpec((t, 16), lambda i: (i, 0))]
        + [pl.BlockSpec(w.shape, lambda i: (0,) * w.ndim) for w in ws],
        out_specs=pl.BlockSpec((t, c), lambda i: (i, 0)),
        out_shape=jax.ShapeDtypeStruct((n, c), jnp.float32),
    )(*([g] * nn), q, pospad, *ws)


def _edge_body2(*refs, c, nn, t):
    _edge_body(refs, c, nn, t)


def _pad16(pos):
    return jnp.pad(pos, ((0, 0), (0, 13)))


def _tblock_pallas(p, x, pospad, nbrT_flat, mode="plain", g=None, pre=None):
    q, tab = _qsv_pallas(p, x, pospad, mode=mode, g=g, pre=pre)
    ge = _sc_gather(tab, nbrT_flat)
    return _edge_pallas(p, ge, q, pospad, K + 1)


def kernel(x, pos, params):
    idx = _build_indices(pos)
    pos_l = [pos]
    for i in range(N_LEVELS):
        pos_l.append(pos_l[i][idx["sel"][i]])
    pp = [_pad16(p) for p in pos_l]
    nbrT = [jnp.transpose(idx["nbr"][i]).reshape(-1) for i in range(N_LEVELS + 1)]
    tdT = [jnp.transpose(idx["td_knn"][i]).reshape(-1) for i in range(N_LEVELS)]
    upT = [jnp.transpose(idx["up_knn"][i]).reshape(-1) for i in range(N_LEVELS)]

    x = _bn_pallas(params["mlp_input"][0], x)
    x = _tblock_pallas(params["t_in"], x, pp[0], nbrT[0])
    outs = [x]
    for i in range(N_LEVELS):
        y = _bn_pallas(params["td"][i][0], x)
        pg = _sc_gather(y, tdT[i])
        x = _tblock_pallas(params["tdown"][i], pg, pp[i + 1], nbrT[i + 1], mode="pool")
        outs.append(x)
    x = _tblock_pallas(params["t_summit"], x, pp[N_LEVELS], nbrT[N_LEVELS],
                       mode="pre", pre=params["mlp_summit"][0])
    for i in range(N_LEVELS - 1, -1, -1):
        xs = _bn_pallas(params["tu"][i]["mlp_sub"][0], x)
        xs_tab = jnp.concatenate([xs, pp[i + 1]], axis=1)
        ug = _sc_gather(xs_tab, upT[i])
        mm = _bn_pallas(params["tu"][i]["mlp"][0], outs[i])
        x = _tblock_pallas(params["tup"][i], mm, pp[i], nbrT[i],
                           mode="interp", g=ug)
    return _head(params["head"], x)


# speedup vs baseline: 9.0265x; 1.0486x over previous
import functools
import jax
import jax.numpy as jnp
import numpy as np
from jax.experimental import pallas as pl
from jax.experimental.pallas import tpu as pltpu

K = 16
DIMS = [32, 64, 128, 256]
N_LEVELS = 3


def _head_body(x_ref, w0, b0, w1, b1, w2, b2, w3, b3, o_ref):
    h = jnp.maximum(jnp.dot(x_ref[...], w0[...], preferred_element_type=jnp.float32) + b0[...], 0.0)
    h = jnp.maximum(jnp.dot(h, w1[...], preferred_element_type=jnp.float32) + b1[...], 0.0)
    h = jnp.maximum(jnp.dot(h, w2[...], preferred_element_type=jnp.float32) + b2[...], 0.0)
    o_ref[...] = jnp.dot(h, w3[...], preferred_element_type=jnp.float32) + b3[...]


def _head(params, x):
    n = x.shape[0]
    tile = 2048
    ws = []
    for p in params:
        ws.append(p["w"])
        ws.append(p["b"].reshape(1, -1))
    grid = (n // tile,)
    return pl.pallas_call(
        _head_body,
        grid=grid,
        in_specs=[pl.BlockSpec((tile, x.shape[1]), lambda i: (i, 0))]
        + [pl.BlockSpec(w.shape, lambda i: (0,) * w.ndim) for w in ws],
        out_specs=pl.BlockSpec((tile, 13), lambda i: (i, 0)),
        out_shape=jax.ShapeDtypeStruct((n, 13), jnp.float32),
    )(x, *ws)


def _knn_body(qp_ref, bT_ref, out_ref, d_scr, k, excl, tq):
    step = pl.program_id(0)
    nb = bT_ref.shape[1]
    nblk = nb // 128
    qp = qp_ref[...]
    bT = bT_ref[...]
    qb = jnp.dot(qp, bT, preferred_element_type=jnp.float32)
    q2 = jnp.sum(qp * qp, axis=1, keepdims=True)
    b2 = jnp.sum(bT * bT, axis=0, keepdims=True)
    d = (q2 + b2) - 2.0 * qb
    if excl:
        col = jax.lax.broadcasted_iota(jnp.int32, (tq, nb), 1)
        row = jax.lax.broadcasted_iota(jnp.int32, (tq, nb), 0) + step * tq
        d = jnp.where(col == row, 1e30, d)

    if nblk > k:
        d_scr[...] = d
        d3 = d.reshape(tq, nblk, 128)
        M = jnp.min(d3, axis=2)
        blk_iota = jax.lax.broadcasted_iota(jnp.int32, (tq, nblk), 1)
        bs = []
        for _ in range(k):
            bmin = jnp.min(M, axis=1, keepdims=True)
            bidx = jnp.min(jnp.where(M == bmin, blk_iota, nblk), axis=1, keepdims=True)
            bs.append(bidx)
            M = jnp.where(blk_iota == bidx, jnp.inf, M)
        B = jnp.concatenate(bs, axis=1)
        CH = 8
        c_q = jax.lax.broadcasted_iota(jnp.int32, (CH, k, CH * nblk), 2) // nblk
        c_b = jax.lax.broadcasted_iota(jnp.int32, (CH, k, CH * nblk), 2) % nblk
        q_i = jax.lax.broadcasted_iota(jnp.int32, (CH, k, CH * nblk), 0)
        w_i = jax.lax.broadcasted_iota(jnp.int32, (CH, k, 128), 2)
        Cs, Gs = [], []
        for qc in range(tq // CH):
            Bc3 = B[qc * CH:(qc + 1) * CH][:, :, None]
            oh = ((c_b == Bc3) & (c_q == q_i)).astype(jnp.float32)
            oh = oh.reshape(CH * k, CH * nblk)
            d2c = d_scr[qc * CH:(qc + 1) * CH, :].reshape(CH * nblk, 128)
            cc = jnp.dot(oh, d2c, preferred_element_type=jnp.float32,
                         precision=jax.lax.Precision.HIGHEST)
            Cs.append(cc.reshape(CH, k, 128))
            Gs.append(Bc3 * 128 + w_i)
        C = jnp.concatenate(Cs, axis=0).reshape(tq, k * 128)
        G = jnp.concatenate(Gs, axis=0).reshape(tq, k * 128)
    else:
        C = d
        G = jax.lax.broadcasted_iota(jnp.int32, (tq, nb), 1)

    outs = []
    for _ in range(k):
        m = jnp.min(C, axis=1, keepdims=True)
        ii = jnp.min(jnp.where(C == m, G, nb), axis=1, keepdims=True)
        outs.append(ii)
        C = jnp.where(G == ii, jnp.inf, C)
    out_ref[...] = jnp.concatenate(outs, axis=1)


def _knn_pallas(query, base, k, exclude_self=False):
    nq, nb = query.shape[0], base.shape[0]
    tq = min(nq, 128)
    qp = jnp.pad(query, ((0, 0), (0, 5)))
    bT = jnp.pad(base, ((0, 0), (0, 5))).T
    grid = (nq // tq,)
    return pl.pallas_call(
        functools.partial(_knn_body, k=k, excl=exclude_self, tq=tq),
        grid=grid,
        in_specs=[
            pl.BlockSpec((tq, 8), lambda i: (i, 0)),
            pl.BlockSpec((8, nb), lambda i: (0, 0)),
        ],
        out_specs=pl.BlockSpec((tq, k), lambda i: (i, 0)),
        out_shape=jax.ShapeDtypeStruct((nq, k), jnp.int32),
        scratch_shapes=[pltpu.VMEM((tq, nb), jnp.float32)],
    )(qp, bT)


def _knn(query, base, k, exclude_self=False, chunk=2048):
    return _knn_pallas(query, base, k, exclude_self=exclude_self)


def _knn_graph(pos, k):
    idx = _knn(pos, pos, k, exclude_self=True)
    self_idx = jnp.arange(pos.shape[0], dtype=idx.dtype)[:, None]
    return jnp.concatenate([idx, self_idx], axis=1)


def _fps_body(xs_ref, ys_ref, zs_ref, xc_ref, yc_ref, zc_ref, sel_ref, nsel):
    x = xs_ref[...]
    y = ys_ref[...]
    z = zs_ref[...]
    rows, lanes = x.shape
    n = rows * lanes
    gidx = jax.lax.broadcasted_iota(jnp.int32, (rows, lanes), 0) * lanes + \
        jax.lax.broadcasted_iota(jnp.int32, (rows, lanes), 1)
    x0 = xs_ref[0, 0]
    y0 = ys_ref[0, 0]
    z0 = zs_ref[0, 0]
    d0 = (x - x0) ** 2 + (y - y0) ** 2 + (z - z0) ** 2
    sel_ref[0:1, 0:1] = jnp.zeros((1, 1), jnp.int32)

    def body(i, d):
        m = jnp.max(d)
        j = jnp.min(jnp.where(d == m, gidx, n))
        sel_ref[pl.ds(i, 1), :] = jnp.full((1, 1), j, jnp.int32)
        xj = xc_ref[pl.ds(j, 1), :][0, 0]
        yj = yc_ref[pl.ds(j, 1), :][0, 0]
        zj = zc_ref[pl.ds(j, 1), :][0, 0]
        dj = (x - xj) ** 2 + (y - yj) ** 2 + (z - zj) ** 2
        return jnp.minimum(d, dj)

    jax.lax.fori_loop(1, nsel, body, d0)


def _fps_pallas(pos):
    n = pos.shape[0]
    nsel = int(np.ceil(0.25 * n))
    rows = n // 128
    xs = pos[:, 0].reshape(rows, 128)
    ys = pos[:, 1].reshape(rows, 128)
    zs = pos[:, 2].reshape(rows, 128)
    cols = [pos[:, 0].reshape(n, 1), pos[:, 1].reshape(n, 1), pos[:, 2].reshape(n, 1)]
    sel = pl.pallas_call(
        functools.partial(_fps_body, nsel=nsel),
        in_specs=[pl.BlockSpec(xs.shape, lambda: (0, 0))] * 3
        + [pl.BlockSpec((n, 1), lambda: (0, 0))] * 3,
        out_specs=pl.BlockSpec((nsel, 1), lambda: (0, 0)),
        out_shape=jax.ShapeDtypeStruct((nsel, 1), jnp.int32),
    )(xs, ys, zs, *cols)
    return sel.reshape(nsel)


def _build_indices(pos):
    pos = jax.lax.stop_gradient(pos)
    pos_l = [pos]
    nbr = [_knn_graph(pos, K)]
    sel, td, up = [], [], []
    for i in range(N_LEVELS):
        s = _fps_pallas(pos_l[i])
        p_sub = pos_l[i][s]
        sel.append(s)
        td.append(_knn(p_sub, pos_l[i], K))
        nbr.append(_knn_graph(p_sub, K))
        pos_l.append(p_sub)
    for i in range(N_LEVELS):
        up.append(_knn(pos_l[i], pos_l[i + 1], 3))
    return {"sel": sel, "td_knn": td, "nbr": nbr, "up_knn": up}


def _sc_gather_seg(table, idx):
    from jax.experimental.pallas import tpu_sc as plsc
    V, D = table.shape
    B = idx.shape[0]
    NW = 32
    bpw = B // NW
    ch = 8
    for c in range(8, min(bpw, 128) + 1, 8):
        if bpw % c == 0:
            ch = c
    nch = bpw // ch
    mesh = plsc.VectorSubcoreMesh(core_axis_name="c", subcore_axis_name="s")

    @functools.partial(
        pl.kernel, mesh=mesh,
        out_type=jax.ShapeDtypeStruct((B, D), jnp.float32),
        scratch_types=[
            pltpu.VMEM((ch,), jnp.int32),
            pltpu.VMEM((ch, D), jnp.float32),
            pltpu.SemaphoreType.DMA,
        ],
    )
    def k(table_hbm, idx_hbm, out_hbm, idx_v, rows_v, sem):
        wid = jax.lax.axis_index("s") * 2 + jax.lax.axis_index("c")
        base = wid * bpw
        for c in range(nch):
            off = base + c * ch
            pltpu.sync_copy(idx_hbm.at[pl.ds(off, ch)], idx_v)
            pltpu.async_copy(table_hbm.at[idx_v], rows_v, sem).wait()
            pltpu.sync_copy(rows_v, out_hbm.at[pl.ds(off, ch)])

    return k(table, idx)


def _sc_gather(table, idx):
    w = table.shape[1]
    if w % 128:
        table = jnp.pad(table, ((0, 0), (0, 128 - w % 128)))
    B = idx.shape[0]
    seg = 69632
    if B <= seg:
        return _sc_gather_seg(table, idx)
    parts = []
    for s in range(0, B, seg):
        parts.append(_sc_gather_seg(table, idx[s:s + seg]))
    return jnp.concatenate(parts, axis=0)


def _dot(a, b):
    return jnp.dot(a, b, preferred_element_type=jnp.float32)


def _bn_body(x_ref, w, b, g, beta, o_ref):
    y = _dot(x_ref[...], w[...]) + b[...]
    mu = jnp.mean(y, axis=0, keepdims=True)
    var = jnp.mean((y - mu) ** 2, axis=0, keepdims=True)
    o_ref[...] = jnp.maximum((y - mu) / jnp.sqrt(var + 1e-5) * g[...] + beta[...], 0.0)


def _bn_pallas(p, x):
    n, ci = x.shape
    co = p["w"].shape[1]
    return pl.pallas_call(
        _bn_body,
        out_shape=jax.ShapeDtypeStruct((n, co), jnp.float32),
    )(x, p["w"], p["b"].reshape(1, -1), p["g"].reshape(1, -1), p["beta"].reshape(1, -1))


def _qsv_body(refs, mode, c, nslab):
    slabs = refs[:nslab]
    pos_ref, wi, bi, wd, ws, wl = refs[nslab:nslab + 6]
    extra = refs[nslab + 6:-2]
    q_ref, tab_ref = refs[-2:]
    cin = wi.shape[0]
    if mode == "pool":
        x = slabs[0][...][:, 0:cin]
        for j in range(1, 16):
            x = jnp.maximum(x, slabs[j][...][:, 0:cin])
    elif mode == "interp":
        x_ref = slabs[0]
        cs = cin
        pos3 = pos_ref[...][:, 0:3]
        num = None
        den = None
        for j in range(3):
            gj = slabs[1 + j][...]
            diff = pos3 - gj[:, cs:cs + 3]
            d2 = jnp.sum(diff * diff, axis=1, keepdims=True)
            d2 = jnp.maximum(d2, 1e-16)
            w8 = 1.0 / d2
            contrib = gj[:, 0:cs] * w8
            num = contrib if num is None else num + contrib
            den = w8 if den is None else den + w8
        x = x_ref[...] + num / den
    elif mode == "pre":
        wp, bp = extra
        x = jnp.maximum(_dot(slabs[0][...], wp[...]) + bp[...], 0.0)
    else:
        x = slabs[0][...]
    h = jnp.maximum(_dot(x, wi[...]) + bi[...], 0.0)
    q_ref[...] = _dot(h, wd[...])
    tab_ref[:, 0:c] = _dot(h, ws[...])
    tab_ref[:, c:2 * c] = _dot(h, wl[...])
    tab_ref[:, 2 * c:2 * c + 16] = pos_ref[...]


def _qsv_body2(*refs, mode, c, nslab):
    _qsv_body(refs, mode, c, nslab)


def _qsv_pallas(p, x, pospad, mode="plain", g=None, pre=None):
    n = pospad.shape[0]
    c = p["w_dst"].shape[1]
    t = min(512, n)
    nt = n // t
    if mode == "pool":
        slab_arrs = [x] * 16
        slab_specs = [pl.BlockSpec((t, x.shape[1]),
                                   functools.partial(lambda i, jj: (jj * nt + i, 0), jj=j))
                      for j in range(16)]
    elif mode == "interp":
        slab_arrs = [x] + [g] * 3
        slab_specs = [pl.BlockSpec((t, x.shape[1]), lambda i: (i, 0))] + \
            [pl.BlockSpec((t, g.shape[1]),
                          functools.partial(lambda i, jj: (jj * nt + i, 0), jj=j))
             for j in range(3)]
    else:
        slab_arrs = [x]
        slab_specs = [pl.BlockSpec((t, x.shape[1]), lambda i: (i, 0))]
    nslab = len(slab_arrs)
    ws = [p["lin_in"]["w"], p["lin_in"]["b"].reshape(1, -1),
          p["w_dst"], p["w_src"], p["w_lin"]]
    extra_in = []
    if mode == "pre":
        extra_in = [pre["w"], pre["b"].reshape(1, -1)]
    return pl.pallas_call(
        functools.partial(_qsv_body2, mode=mode, c=c, nslab=nslab),
        grid=(nt,),
        in_specs=slab_specs
        + [pl.BlockSpec((t, 16), lambda i: (i, 0))]
        + [pl.BlockSpec(w.shape, lambda i: (0,) * w.ndim) for w in ws]
        + [pl.BlockSpec(w.shape, lambda i: (0,) * w.ndim) for w in extra_in],
        out_specs=(pl.BlockSpec((t, c), lambda i: (i, 0)),
                   pl.BlockSpec((t, 2 * c + 16), lambda i: (i, 0))),
        out_shape=(jax.ShapeDtypeStruct((n, c), jnp.float32),
                   jax.ShapeDtypeStruct((n, 2 * c + 16), jnp.float32)),
    )(*slab_arrs, pospad, *ws, *extra_in)


def _edge_body(refs, c, nn, t):
    gs = refs[:nn]
    q_ref, pos_ref = refs[nn], refs[nn + 1]
    w1, b1, w2, b2, a1, ab1, a2, ab2, wo, bo = refs[nn + 2:nn + 12]
    o_ref = refs[nn + 12]
    pos3 = pos_ref[...][:, 0:3]
    q = q_ref[...]

    def delta_j(j):
        gj = gs[j]
        pd = pos3 - gj[:, 2 * c:2 * c + 3]
        h = jnp.maximum(_dot(pd, w1[...]) + b1[...], 0.0)
        return jnp.maximum(_dot(h, w2[...]) + b2[...], 0.0)

    avs = []
    for j in range(nn):
        a = q - gs[j][:, 0:c] + delta_j(j)
        h = jnp.maximum(_dot(a, a1[...]) + ab1[...], 0.0)
        avs.append(jnp.maximum(_dot(h, a2[...]) + ab2[...], 0.0))
    mx = avs[0]
    for j in range(1, nn):
        mx = jnp.maximum(mx, avs[j])
    ssum = None
    for j in range(nn):
        e = jnp.exp(avs[j] - mx)
        ssum = e if ssum is None else ssum + e
    acc = None
    for j in range(nn):
        term = (jnp.exp(avs[j] - mx) / ssum) * (gs[j][:, c:2 * c] + delta_j(j))
        acc = term if acc is None else acc + term
    o_ref[...] = jnp.maximum(_dot(acc, wo[...]) + bo[...], 0.0)


def _edge_pallas(p, g, q, pospad, nn):
    n, c = q.shape
    wd = g.shape[1]
    row_bytes = nn * wd * 8 + 5 * nn * c * 4
    t = 64
    for cand in (1024, 512, 256, 128, 64):
        if n % cand == 0 and cand * row_bytes <= 16_000_000:
            t = min(cand, n)
            break
    grid = (n // t,)
    nt = n // t
    g_specs = [pl.BlockSpec((t, wd), functools.partial(lambda i, jj: (jj * nt + i, 0), jj=j))
               for j in range(nn)]
    ws = [p["pos_nn"][0]["w"], p["pos_nn"][0]["b"].reshape(1, -1),
          p["pos_nn"][1]["w"], p["pos_nn"][1]["b"].reshape(1, -1),
          p["attn_nn"][0]["w"], p["attn_nn"][0]["b"].reshape(1, -1),
          p["attn_nn"][1]["w"], p["attn_nn"][1]["b"].reshape(1, -1),
          p["lin_out"]["w"], p["lin_out"]["b"].reshape(1, -1)]
    return pl.pallas_call(
        functools.partial(_edge_body2, c=c, nn=nn, t=t),
        grid=grid,
        in_specs=g_specs
        + [pl.BlockSpec((t, c), lambda i: (i, 0)),
           pl.BlockSpec((t, 16), lambda i: (i, 0))]
        + [pl.BlockSpec(w.shape, lambda i: (0,) * w.ndim) for w in ws],
        out_specs=pl.BlockSpec((t, c), lambda i: (i, 0)),
        out_shape=jax.ShapeDtypeStruct((n, c), jnp.float32),
    )(*([g] * nn), q, pospad, *ws)


def _edge_body2(*refs, c, nn, t):
    _edge_body(refs, c, nn, t)


def _pad16(pos):
    return jnp.pad(pos, ((0, 0), (0, 13)))


def _tblock_pallas(p, x, pospad, nbrT_flat, mode="plain", g=None, pre=None):
    q, tab = _qsv_pallas(p, x, pospad, mode=mode, g=g, pre=pre)
    ge = _sc_gather(tab, nbrT_flat)
    return _edge_pallas(p, ge, q, pospad, K + 1)


def kernel(x, pos, params):
    idx = _build_indices(pos)
    pos_l = [pos]
    for i in range(N_LEVELS):
        pos_l.append(pos_l[i][idx["sel"][i]])
    pp = [_pad16(p) for p in pos_l]
    nbrT = [jnp.transpose(idx["nbr"][i]).reshape(-1) for i in range(N_LEVELS + 1)]
    tdT = [jnp.transpose(idx["td_knn"][i]).reshape(-1) for i in range(N_LEVELS)]
    upT = [jnp.transpose(idx["up_knn"][i]).reshape(-1) for i in range(N_LEVELS)]

    x = _bn_pallas(params["mlp_input"][0], x)
    x = _tblock_pallas(params["t_in"], x, pp[0], nbrT[0])
    outs = [x]
    for i in range(N_LEVELS):
        y = _bn_pallas(params["td"][i][0], x)
        pg = _sc_gather(y, tdT[i])
        x = _tblock_pallas(params["tdown"][i], pg, pp[i + 1], nbrT[i + 1], mode="pool")
        outs.append(x)
    x = _tblock_pallas(params["t_summit"], x, pp[N_LEVELS], nbrT[N_LEVELS],
                       mode="pre", pre=params["mlp_summit"][0])
    for i in range(N_LEVELS - 1, -1, -1):
        xs = _bn_pallas(params["tu"][i]["mlp_sub"][0], x)
        xs_tab = jnp.concatenate([xs, pp[i + 1]], axis=1)
        ug = _sc_gather(xs_tab, upT[i])
        mm = _bn_pallas(params["tu"][i]["mlp"][0], outs[i])
        x = _tblock_pallas(params["tup"][i], mm, pp[i], nbrT[i],
                           mode="interp", g=ug)
    return _head(params["head"], x)
